# baseline (speedup 1.0000x reference)
import functools

import jax
import jax.numpy as jnp
from jax import lax
from jax.experimental import pallas as pl
from jax.experimental.pallas import tpu as pltpu
from jax.experimental.pallas import tpu_sc as plsc

BATCH = 16384
NF = 16
NC = 2
NS = 16
L = 16
NW = NC * NS
BPW = BATCH // NW
Y_LO = 0.0
Y_HI = 10.1

_mesh = plsc.VectorSubcoreMesh(core_axis_name="c", subcore_axis_name="s")


@functools.partial(
    pl.kernel,
    out_type=jax.ShapeDtypeStruct((BATCH,), jnp.float32),
    mesh=_mesh,
    scratch_types=[
        pltpu.VMEM((BPW,), jnp.int32),
        pltpu.VMEM((BPW,), jnp.int32),
        pltpu.VMEM((BPW, NF), jnp.float32),
        pltpu.VMEM((BPW, NF), jnp.float32),
        pltpu.VMEM((BPW,), jnp.float32),
        pltpu.VMEM((BPW,), jnp.float32),
        pltpu.VMEM((BPW,), jnp.float32),
        pltpu.SemaphoreType.DMA,
    ],
    compiler_params=pltpu.CompilerParams(
        needs_layout_passes=False, use_tc_tiling_on_sc=False),
)
def _sc_dot(uidx_hbm, bidx_hbm, uf_hbm, bf_hbm, ub_hbm, bb_hbm, out_hbm,
            uidx_v, bidx_v, urows_v, brows_v, ubias_v, bbias_v, out_v, sem):
    wid = lax.axis_index("s") * NC + lax.axis_index("c")
    base = pl.multiple_of(wid * BPW, BPW)

    pltpu.sync_copy(uidx_hbm.at[pl.ds(base, BPW)], uidx_v)
    pltpu.sync_copy(bidx_hbm.at[pl.ds(base, BPW)], bidx_v)

    copies = [
        pltpu.async_copy(uf_hbm.at[uidx_v], urows_v, sem),
        pltpu.async_copy(bf_hbm.at[bidx_v], brows_v, sem),
        pltpu.async_copy(ub_hbm.at[uidx_v], ubias_v, sem),
        pltpu.async_copy(bb_hbm.at[bidx_v], bbias_v, sem),
    ]
    for cp in copies:
        cp.wait()

    lanes = lax.iota(jnp.int32, L)

    def group(g, carry):
        row = g * L + lanes
        acc = jnp.zeros((L,), jnp.float32)
        for k in range(NF):
            col = (lanes + k) & (NF - 1)
            u = plsc.load_gather(urows_v, [row, col])
            b = plsc.load_gather(brows_v, [row, col])
            acc = acc + u * b
        off = pl.multiple_of(g * L, L)
        acc = acc + ubias_v[pl.ds(off, L)] + bbias_v[pl.ds(off, L)]
        out_v[pl.ds(off, L)] = (Y_HI - Y_LO) / (1.0 + jnp.exp(-acc)) + Y_LO
        return carry

    lax.fori_loop(0, BPW // L, group, 0)

    pltpu.sync_copy(out_v, out_hbm.at[pl.ds(base, BPW)])


def kernel(x, users_factors, books_factors, users_bias, books_bias):
    uidx = x[:, 0]
    bidx = x[:, 1]
    uf = users_factors.astype(jnp.bfloat16).astype(jnp.float32)
    bf = books_factors.astype(jnp.bfloat16).astype(jnp.float32)
    out = _sc_dot(uidx, bidx, uf, bf,
                  users_bias.reshape(-1), books_bias.reshape(-1))
    return out.reshape(BATCH, 1)


# speedup vs baseline: 2.0116x; 2.0116x over previous
import functools

import jax
import jax.numpy as jnp
from jax import lax
from jax.experimental import pallas as pl
from jax.experimental.pallas import tpu as pltpu
from jax.experimental.pallas import tpu_sc as plsc

BATCH = 16384
NF = 16
NV = 1000000
BLK = 2048
NBLK = (NV + BLK - 1) // BLK
SEG = NF * BLK
FLAT = NBLK * SEG
NC = 2
NS = 16
L = 16
NW = NC * NS
BPW = BATCH // NW
Y_LO = 0.0
Y_HI = 10.1


def _detile_body(u_ref, b_ref, uo_ref, bo_ref):
    for k in range(NF):
        uo_ref[pl.ds(k * BLK, BLK)] = u_ref[k, :]
        bo_ref[pl.ds(k * BLK, BLK)] = b_ref[k, :]


_detile = pl.pallas_call(
    _detile_body,
    grid=(NBLK,),
    in_specs=[
        pl.BlockSpec((NF, BLK), lambda i: (0, i)),
        pl.BlockSpec((NF, BLK), lambda i: (0, i)),
    ],
    out_specs=[
        pl.BlockSpec((SEG,), lambda i: (i,)),
        pl.BlockSpec((SEG,), lambda i: (i,)),
    ],
    out_shape=[
        jax.ShapeDtypeStruct((FLAT,), jnp.float32),
        jax.ShapeDtypeStruct((FLAT,), jnp.float32),
    ],
)

_mesh = plsc.VectorSubcoreMesh(core_axis_name="c", subcore_axis_name="s")


@functools.partial(
    pl.kernel,
    out_type=jax.ShapeDtypeStruct((BATCH,), jnp.float32),
    mesh=_mesh,
    scratch_types=[
        pltpu.VMEM((BPW,), jnp.int32),
        pltpu.VMEM((BPW,), jnp.int32),
        pltpu.VMEM((NF, BPW), jnp.int32),
        pltpu.VMEM((NF, BPW), jnp.int32),
        pltpu.VMEM((NF, BPW), jnp.float32),
        pltpu.VMEM((NF, BPW), jnp.float32),
        pltpu.VMEM((BPW,), jnp.float32),
        pltpu.VMEM((BPW,), jnp.float32),
        pltpu.VMEM((BPW,), jnp.float32),
        pltpu.SemaphoreType.DMA,
    ],
    compiler_params=pltpu.CompilerParams(
        needs_layout_passes=False, use_tc_tiling_on_sc=False),
)
def _sc_dot(uidx_hbm, bidx_hbm, uflat_hbm, bflat_hbm, ub_hbm, bb_hbm, out_hbm,
            uidx_v, bidx_v, ufi_v, bfi_v, urows_v, brows_v,
            ubias_v, bbias_v, out_v, sem):
    wid = lax.axis_index("s") * NC + lax.axis_index("c")
    base = pl.multiple_of(wid * BPW, BPW)

    pltpu.sync_copy(uidx_hbm.at[pl.ds(base, BPW)], uidx_v)
    pltpu.sync_copy(bidx_hbm.at[pl.ds(base, BPW)], bidx_v)

    def mkidx(g, carry):
        off = pl.multiple_of(g * L, L)
        ur = uidx_v[pl.ds(off, L)]
        br = bidx_v[pl.ds(off, L)]
        ubase = ((ur >> 11) << 15) + (ur & (BLK - 1))
        bbase = ((br >> 11) << 15) + (br & (BLK - 1))
        for k in range(NF):
            ufi_v[k, pl.ds(off, L)] = ubase + (k * BLK)
            bfi_v[k, pl.ds(off, L)] = bbase + (k * BLK)
        return carry

    lax.fori_loop(0, BPW // L, mkidx, 0)

    copies = [
        pltpu.async_copy(ub_hbm.at[uidx_v], ubias_v, sem),
        pltpu.async_copy(bb_hbm.at[bidx_v], bbias_v, sem),
    ]
    for k in range(NF):
        copies.append(
            pltpu.async_copy(uflat_hbm.at[ufi_v.at[k]], urows_v.at[k], sem))
        copies.append(
            pltpu.async_copy(bflat_hbm.at[bfi_v.at[k]], brows_v.at[k], sem))
    for cp in copies:
        cp.wait()

    def group(g, carry):
        off = pl.multiple_of(g * L, L)
        acc = ubias_v[pl.ds(off, L)] + bbias_v[pl.ds(off, L)]
        for k in range(NF):
            acc = acc + urows_v[k, pl.ds(off, L)] * brows_v[k, pl.ds(off, L)]
        out_v[pl.ds(off, L)] = (Y_HI - Y_LO) / (1.0 + jnp.exp(-acc)) + Y_LO
        return carry

    lax.fori_loop(0, BPW // L, group, 0)

    pltpu.sync_copy(out_v, out_hbm.at[pl.ds(base, BPW)])


def kernel(x, users_factors, books_factors, users_bias, books_bias):
    uidx = x[:, 0]
    bidx = x[:, 1]
    uflat, bflat = _detile(users_factors.T, books_factors.T)
    out = _sc_dot(uidx, bidx, uflat, bflat,
                  users_bias.reshape(-1), books_bias.reshape(-1))
    return out.reshape(BATCH, 1)


# speedup vs baseline: 3.9141x; 1.9457x over previous
import functools

import jax
import jax.numpy as jnp
from jax import lax
from jax.experimental import pallas as pl
from jax.experimental.pallas import tpu as pltpu
from jax.experimental.pallas import tpu_sc as plsc

BATCH = 16384
NF = 16
NV = 1000000
BLK = 16384
NBLK = (NV + BLK - 1) // BLK
SEG = NF * BLK
FLAT = NBLK * SEG
NC = 2
NS = 16
L = 16
NW = NC * NS
BPW = BATCH // NW
Y_LO = 0.0
Y_HI = 10.1


def _detile_body(u_ref, b_ref, uo_ref, bo_ref):
    for k in range(NF):
        uo_ref[pl.ds(k * BLK, BLK)] = u_ref[k, :]
        bo_ref[pl.ds(k * BLK, BLK)] = b_ref[k, :]


_detile = pl.pallas_call(
    _detile_body,
    grid=(NBLK,),
    in_specs=[
        pl.BlockSpec((NF, BLK), lambda i: (0, i)),
        pl.BlockSpec((NF, BLK), lambda i: (0, i)),
    ],
    out_specs=[
        pl.BlockSpec((SEG,), lambda i: (i,)),
        pl.BlockSpec((SEG,), lambda i: (i,)),
    ],
    out_shape=[
        jax.ShapeDtypeStruct((FLAT,), jnp.float32),
        jax.ShapeDtypeStruct((FLAT,), jnp.float32),
    ],
)

_mesh = plsc.VectorSubcoreMesh(core_axis_name="c", subcore_axis_name="s")


@functools.partial(
    pl.kernel,
    out_type=jax.ShapeDtypeStruct((BATCH,), jnp.float32),
    mesh=_mesh,
    scratch_types=[
        pltpu.VMEM((BPW,), jnp.int32),
        pltpu.VMEM((BPW,), jnp.int32),
        pltpu.VMEM((NF, BPW), jnp.int32),
        pltpu.VMEM((NF, BPW), jnp.int32),
        pltpu.VMEM((NF, BPW), jnp.float32),
        pltpu.VMEM((NF, BPW), jnp.float32),
        pltpu.VMEM((BPW,), jnp.float32),
        pltpu.VMEM((BPW,), jnp.float32),
        pltpu.VMEM((BPW,), jnp.float32),
        pltpu.SemaphoreType.DMA,
    ],
    compiler_params=pltpu.CompilerParams(
        needs_layout_passes=False, use_tc_tiling_on_sc=False),
)
def _sc_dot(uidx_hbm, bidx_hbm, uflat_hbm, bflat_hbm, ub_hbm, bb_hbm, out_hbm,
            uidx_v, bidx_v, ufi_v, bfi_v, urows_v, brows_v,
            ubias_v, bbias_v, out_v, sem):
    wid = lax.axis_index("s") * NC + lax.axis_index("c")
    base = pl.multiple_of(wid * BPW, BPW)

    pltpu.sync_copy(uidx_hbm.at[pl.ds(base, BPW)], uidx_v)
    pltpu.sync_copy(bidx_hbm.at[pl.ds(base, BPW)], bidx_v)

    def mkidx(g, carry):
        off = pl.multiple_of(g * L, L)
        ur = uidx_v[pl.ds(off, L)]
        br = bidx_v[pl.ds(off, L)]
        ubase = ((ur >> 14) << 18) + (ur & (BLK - 1))
        bbase = ((br >> 14) << 18) + (br & (BLK - 1))
        for k in range(NF):
            ufi_v[k, pl.ds(off, L)] = ubase + (k * BLK)
            bfi_v[k, pl.ds(off, L)] = bbase + (k * BLK)
        return carry

    lax.fori_loop(0, BPW // L, mkidx, 0)

    copies = [
        pltpu.async_copy(ub_hbm.at[uidx_v], ubias_v, sem),
        pltpu.async_copy(bb_hbm.at[bidx_v], bbias_v, sem),
    ]
    for k in range(NF):
        copies.append(
            pltpu.async_copy(uflat_hbm.at[ufi_v.at[k]], urows_v.at[k], sem))
        copies.append(
            pltpu.async_copy(bflat_hbm.at[bfi_v.at[k]], brows_v.at[k], sem))
    for cp in copies:
        cp.wait()

    def group(g, carry):
        off = pl.multiple_of(g * L, L)
        acc = ubias_v[pl.ds(off, L)] + bbias_v[pl.ds(off, L)]
        for k in range(NF):
            acc = acc + urows_v[k, pl.ds(off, L)] * brows_v[k, pl.ds(off, L)]
        out_v[pl.ds(off, L)] = (Y_HI - Y_LO) / (1.0 + jnp.exp(-acc)) + Y_LO
        return carry

    lax.fori_loop(0, BPW // L, group, 0)

    pltpu.sync_copy(out_v, out_hbm.at[pl.ds(base, BPW)])


def kernel(x, users_factors, books_factors, users_bias, books_bias):
    uidx = x[:, 0]
    bidx = x[:, 1]
    uflat, bflat = _detile(users_factors.T, books_factors.T)
    out = _sc_dot(uidx, bidx, uflat, bflat,
                  users_bias.reshape(-1), books_bias.reshape(-1))
    return out.reshape(BATCH, 1)


# speedup vs baseline: 4.0803x; 1.0425x over previous
import functools

import jax
import jax.numpy as jnp
from jax import lax
from jax.experimental import pallas as pl
from jax.experimental.pallas import tpu as pltpu
from jax.experimental.pallas import tpu_sc as plsc

BATCH = 16384
NF = 16
NV = 1000000
BLK = 65536
NBLK = (NV + BLK - 1) // BLK
SEG = NF * BLK
FLAT = NBLK * SEG
NC = 2
NS = 16
L = 16
NW = NC * NS
BPW = BATCH // NW
Y_LO = 0.0
Y_HI = 10.1


def _detile_body(u_ref, b_ref, uo_ref, bo_ref):
    for k in range(NF):
        uo_ref[pl.ds(k * BLK, BLK)] = u_ref[k, :]
        bo_ref[pl.ds(k * BLK, BLK)] = b_ref[k, :]


_detile = pl.pallas_call(
    _detile_body,
    grid=(NBLK,),
    in_specs=[
        pl.BlockSpec((NF, BLK), lambda i: (0, i)),
        pl.BlockSpec((NF, BLK), lambda i: (0, i)),
    ],
    out_specs=[
        pl.BlockSpec((SEG,), lambda i: (i,)),
        pl.BlockSpec((SEG,), lambda i: (i,)),
    ],
    out_shape=[
        jax.ShapeDtypeStruct((FLAT,), jnp.float32),
        jax.ShapeDtypeStruct((FLAT,), jnp.float32),
    ],
)

_mesh = plsc.VectorSubcoreMesh(core_axis_name="c", subcore_axis_name="s")


@functools.partial(
    pl.kernel,
    out_type=jax.ShapeDtypeStruct((BATCH,), jnp.float32),
    mesh=_mesh,
    scratch_types=[
        pltpu.VMEM((BPW,), jnp.int32),
        pltpu.VMEM((BPW,), jnp.int32),
        pltpu.VMEM((NF, BPW), jnp.int32),
        pltpu.VMEM((NF, BPW), jnp.int32),
        pltpu.VMEM((NF, BPW), jnp.float32),
        pltpu.VMEM((NF, BPW), jnp.float32),
        pltpu.VMEM((BPW,), jnp.float32),
        pltpu.VMEM((BPW,), jnp.float32),
        pltpu.VMEM((BPW,), jnp.float32),
        pltpu.SemaphoreType.DMA,
    ],
    compiler_params=pltpu.CompilerParams(
        needs_layout_passes=False, use_tc_tiling_on_sc=False),
)
def _sc_dot(uidx_hbm, bidx_hbm, uflat_hbm, bflat_hbm, ub_hbm, bb_hbm, out_hbm,
            uidx_v, bidx_v, ufi_v, bfi_v, urows_v, brows_v,
            ubias_v, bbias_v, out_v, sem):
    wid = lax.axis_index("s") * NC + lax.axis_index("c")
    base = pl.multiple_of(wid * BPW, BPW)

    pltpu.sync_copy(uidx_hbm.at[pl.ds(base, BPW)], uidx_v)
    pltpu.sync_copy(bidx_hbm.at[pl.ds(base, BPW)], bidx_v)

    def mkidx(g, carry):
        off = pl.multiple_of(g * L, L)
        ur = uidx_v[pl.ds(off, L)]
        br = bidx_v[pl.ds(off, L)]
        ubase = ((ur >> 16) << 20) + (ur & (BLK - 1))
        bbase = ((br >> 16) << 20) + (br & (BLK - 1))
        for k in range(NF):
            ufi_v[k, pl.ds(off, L)] = ubase + (k * BLK)
            bfi_v[k, pl.ds(off, L)] = bbase + (k * BLK)
        return carry

    lax.fori_loop(0, BPW // L, mkidx, 0)

    copies = [
        pltpu.async_copy(ub_hbm.at[uidx_v], ubias_v, sem),
        pltpu.async_copy(bb_hbm.at[bidx_v], bbias_v, sem),
    ]
    for k in range(NF):
        copies.append(
            pltpu.async_copy(uflat_hbm.at[ufi_v.at[k]], urows_v.at[k], sem))
        copies.append(
            pltpu.async_copy(bflat_hbm.at[bfi_v.at[k]], brows_v.at[k], sem))
    for cp in copies:
        cp.wait()

    def group(g, carry):
        off = pl.multiple_of(g * L, L)
        acc = ubias_v[pl.ds(off, L)] + bbias_v[pl.ds(off, L)]
        for k in range(NF):
            acc = acc + urows_v[k, pl.ds(off, L)] * brows_v[k, pl.ds(off, L)]
        out_v[pl.ds(off, L)] = (Y_HI - Y_LO) / (1.0 + jnp.exp(-acc)) + Y_LO
        return carry

    lax.fori_loop(0, BPW // L, group, 0)

    pltpu.sync_copy(out_v, out_hbm.at[pl.ds(base, BPW)])


def kernel(x, users_factors, books_factors, users_bias, books_bias):
    uidx = x[:, 0]
    bidx = x[:, 1]
    uflat, bflat = _detile(users_factors.T, books_factors.T)
    out = _sc_dot(uidx, bidx, uflat, bflat,
                  users_bias.reshape(-1), books_bias.reshape(-1))
    return out.reshape(BATCH, 1)
